# baseline (speedup 1.0000x reference)
import functools

import jax
import jax.numpy as jnp
from jax import lax
from jax.experimental import pallas as pl
from jax.experimental.pallas import tpu as pltpu
from jax.experimental.pallas import tpu_sc as plsc

B = 16384
D = 128
NC = 2
NS = 16
NW = NC * NS
BPW = B // NW
CHUNK = 128
NCHUNK = BPW // CHUNK

_mesh = plsc.VectorSubcoreMesh(core_axis_name="c", subcore_axis_name="s")

_GDN = lax.GatherDimensionNumbers(
    offset_dims=(), collapsed_slice_dims=(0,), start_index_map=(0,)
)


def _perm(v, idx):
    return lax.gather(
        v,
        idx[:, None],
        dimension_numbers=_GDN,
        slice_sizes=(1,),
        mode=lax.GatherScatterMode.PROMISE_IN_BOUNDS,
    )


def _transpose16(vs, lane):
    for s in (1, 2, 4, 8):
        msk = (lane & s) == 0
        x = lane ^ s
        new = list(vs)
        for i in range(16):
            if i & s:
                continue
            j = i + s
            a, b = vs[i], vs[j]
            new[i] = jnp.where(msk, a, _perm(b, x))
            new[j] = jnp.where(msk, _perm(a, x), b)
        vs = new
    return vs


@functools.partial(
    pl.kernel,
    mesh=_mesh,
    out_type=jax.ShapeDtypeStruct((D, B), jnp.float32),
    compiler_params=pltpu.CompilerParams(needs_layout_passes=False),
    scratch_types=[
        pltpu.VMEM((BPW,), jnp.int32),
        pltpu.VMEM((CHUNK, D), jnp.float32),
        pltpu.VMEM((CHUNK, D), jnp.float32),
        pltpu.VMEM((D, BPW), jnp.float32),
        pltpu.SemaphoreType.DMA,
        pltpu.SemaphoreType.DMA,
    ],
)
def _sc_gather_t(table_hbm, idx_hbm, out_hbm, idx_v, rows_a, rows_b, t_v,
                 sem_a, sem_b):
    wid = lax.axis_index("s") * NC + lax.axis_index("c")
    base = wid * BPW
    pltpu.sync_copy(idx_hbm.at[pl.ds(base, BPW)], idx_v)

    lane = lax.iota(jnp.int32, 16)

    def transpose_chunk(rows_v, c):
        @plsc.parallel_loop(0, (CHUNK // 16) * (D // 16), 1, unroll=2)
        def bb_loop(bb):
            bi = bb // (D // 16)
            bj = bb % (D // 16)
            r0 = bi * 16
            vs = [rows_v[r0 + r, pl.ds(bj * 16, 16)] for r in range(16)]
            ws = _transpose16(vs, lane)
            for r in range(16):
                t_v[bj * 16 + r, pl.ds(c * CHUNK + r0, 16)] = ws[r]

    def gather_chunk(c, rows_v, sem):
        pltpu.async_copy(
            table_hbm.at[idx_v.at[pl.ds(c * CHUNK, CHUNK)]], rows_v, sem
        )

    gather_chunk(0, rows_a, sem_a)

    def do_pair(cc, _):
        c0 = cc * 2
        gather_chunk(c0 + 1, rows_b, sem_b)
        pltpu.make_async_copy(
            table_hbm.at[idx_v.at[pl.ds(0, CHUNK)]], rows_a, sem_a
        ).wait()
        transpose_chunk(rows_a, c0)

        @pl.when(cc + 1 < NCHUNK // 2)
        def _():
            gather_chunk(c0 + 2, rows_a, sem_a)

        pltpu.make_async_copy(
            table_hbm.at[idx_v.at[pl.ds(0, CHUNK)]], rows_b, sem_b
        ).wait()
        transpose_chunk(rows_b, c0 + 1)
        return 0

    lax.fori_loop(0, NCHUNK // 2, do_pair, 0)
    pltpu.sync_copy(t_v, out_hbm.at[:, pl.ds(base, BPW)])


def kernel(task_index, task_index_to_mask):
    idx = task_index.reshape(B).astype(jnp.int32)
    return _sc_gather_t(task_index_to_mask, idx)[:, :, None]
